# baseline (speedup 1.0000x reference)
import jax
import jax.numpy as jnp
from jax.experimental import pallas as pl
from jax.experimental.pallas import tpu as pltpu

N = 4096
D_IN = 512
D_OUT = 512
BI = 512

_SIZES = [512] * 7 + [256, 128, 128]
_OFFS = [sum(_SIZES[:i]) for i in range(len(_SIZES))]
NS = len(_SIZES)


def _fused_kernel(x_hbm, w_hbm, adj_hbm, o_hbm,
                  x_v, w_v, h_v, a0, a1, o0, o1,
                  sx, sw, sa0, sa1, so0, so1):
    abuf = [a0, a1]
    asem = [sa0, sa1]
    obuf = [o0, o1]
    osem = [so0, so1]

    cx = pltpu.make_async_copy(x_hbm, x_v, sx)
    cx.start()
    cw = pltpu.make_async_copy(w_hbm, w_v, sw)
    cw.start()
    cx.wait()
    cw.wait()

    def adj_copy(i, b):
        return pltpu.make_async_copy(
            adj_hbm.at[pl.ds(_OFFS[i], _SIZES[i]), :],
            abuf[b].at[pl.ds(0, _SIZES[i]), :],
            asem[b],
        )

    def out_copy(i, b):
        return pltpu.make_async_copy(
            obuf[b].at[pl.ds(0, _SIZES[i]), :],
            o_hbm.at[pl.ds(_OFFS[i], _SIZES[i]), :],
            osem[b],
        )

    adj_copy(0, 0).start()
    adj_copy(1, 1).start()

    h_v[...] = jnp.dot(
        x_v[...], w_v[...], preferred_element_type=jnp.float32
    ).astype(jnp.bfloat16)

    for i in range(NS):
        b = i % 2
        adj_copy(i, b).wait()
        a = abuf[b][pl.ds(0, _SIZES[i]), :]
        deg = jnp.sum(a, axis=1, keepdims=True)
        acc = jnp.dot(
            a.astype(jnp.bfloat16), h_v[...],
            preferred_element_type=jnp.float32,
        )
        if i >= 2:
            out_copy(i - 2, b).wait()
        obuf[b][pl.ds(0, _SIZES[i]), :] = acc / deg
        out_copy(i, b).start()
        if i + 2 < NS:
            adj_copy(i + 2, b).start()

    for i in (NS - 2, NS - 1):
        out_copy(i, i % 2).wait()


@jax.jit
def kernel(input, adj, W):
    return pl.pallas_call(
        _fused_kernel,
        in_specs=[
            pl.BlockSpec(memory_space=pltpu.MemorySpace.HBM),
            pl.BlockSpec(memory_space=pltpu.MemorySpace.HBM),
            pl.BlockSpec(memory_space=pltpu.MemorySpace.HBM),
        ],
        out_specs=pl.BlockSpec(memory_space=pltpu.MemorySpace.HBM),
        out_shape=jax.ShapeDtypeStruct((N, D_OUT), jnp.float32),
        scratch_shapes=[
            pltpu.VMEM((N, D_IN), jnp.float32),
            pltpu.VMEM((D_IN, D_OUT), jnp.float32),
            pltpu.VMEM((N, D_OUT), jnp.bfloat16),
            pltpu.VMEM((BI, N), jnp.float32),
            pltpu.VMEM((BI, N), jnp.float32),
            pltpu.VMEM((BI, D_OUT), jnp.float32),
            pltpu.VMEM((BI, D_OUT), jnp.float32),
            pltpu.SemaphoreType.DMA,
            pltpu.SemaphoreType.DMA,
            pltpu.SemaphoreType.DMA,
            pltpu.SemaphoreType.DMA,
            pltpu.SemaphoreType.DMA,
            pltpu.SemaphoreType.DMA,
        ],
    )(input, W, adj)


# speedup vs baseline: 1.0280x; 1.0280x over previous
import jax
import jax.numpy as jnp
from jax.experimental import pallas as pl
from jax.experimental.pallas import tpu as pltpu

N = 4096
D_IN = 512
D_OUT = 512
BI = 512
NI = N // BI


def _fused_kernel(x_hbm, w_hbm, adj_hbm, o_hbm,
                  x_v, w_v, h_v, a0, a1, o0, o1,
                  sx, sw, sa0, sa1, so0, so1):
    abuf = [a0, a1]
    asem = [sa0, sa1]
    obuf = [o0, o1]
    osem = [so0, so1]

    cx = pltpu.make_async_copy(x_hbm, x_v, sx)
    cx.start()
    cw = pltpu.make_async_copy(w_hbm, w_v, sw)
    cw.start()
    cx.wait()
    cw.wait()

    for j in range(min(2, NI)):
        pltpu.make_async_copy(
            adj_hbm.at[pl.ds(j * BI, BI), :], abuf[j], asem[j]
        ).start()

    h_v[...] = jnp.dot(
        x_v[...], w_v[...], preferred_element_type=jnp.float32
    ).astype(jnp.bfloat16)

    for i in range(NI):
        b = i % 2
        pltpu.make_async_copy(
            adj_hbm.at[pl.ds(i * BI, BI), :], abuf[b], asem[b]
        ).wait()
        a = abuf[b][...]
        deg = jnp.sum(a, axis=1, keepdims=True)
        acc = jnp.dot(
            a.astype(jnp.bfloat16), h_v[...],
            preferred_element_type=jnp.float32,
        )
        if i >= 2:
            pltpu.make_async_copy(
                obuf[b], o_hbm.at[pl.ds((i - 2) * BI, BI), :], osem[b]
            ).wait()
        obuf[b][...] = acc / deg
        pltpu.make_async_copy(
            obuf[b], o_hbm.at[pl.ds(i * BI, BI), :], osem[b]
        ).start()
        if i + 2 < NI:
            pltpu.make_async_copy(
                adj_hbm.at[pl.ds((i + 2) * BI, BI), :], abuf[b], asem[b]
            ).start()

    for i in (NI - 2, NI - 1):
        b = i % 2
        pltpu.make_async_copy(
            obuf[b], o_hbm.at[pl.ds(i * BI, BI), :], osem[b]
        ).wait()


@jax.jit
def kernel(input, adj, W):
    return pl.pallas_call(
        _fused_kernel,
        in_specs=[
            pl.BlockSpec(memory_space=pltpu.MemorySpace.HBM),
            pl.BlockSpec(memory_space=pltpu.MemorySpace.HBM),
            pl.BlockSpec(memory_space=pltpu.MemorySpace.HBM),
        ],
        out_specs=pl.BlockSpec(memory_space=pltpu.MemorySpace.HBM),
        out_shape=jax.ShapeDtypeStruct((N, D_OUT), jnp.float32),
        scratch_shapes=[
            pltpu.VMEM((N, D_IN), jnp.float32),
            pltpu.VMEM((D_IN, D_OUT), jnp.float32),
            pltpu.VMEM((N, D_OUT), jnp.bfloat16),
            pltpu.VMEM((BI, N), jnp.float32),
            pltpu.VMEM((BI, N), jnp.float32),
            pltpu.VMEM((BI, D_OUT), jnp.float32),
            pltpu.VMEM((BI, D_OUT), jnp.float32),
            pltpu.SemaphoreType.DMA,
            pltpu.SemaphoreType.DMA,
            pltpu.SemaphoreType.DMA,
            pltpu.SemaphoreType.DMA,
            pltpu.SemaphoreType.DMA,
            pltpu.SemaphoreType.DMA,
        ],
    )(input, W, adj)
